# Initial kernel scaffold; baseline (speedup 1.0000x reference)
import jax
import jax.numpy as jnp
from jax.experimental import pallas as pl


def kernel(updates, mask):
    raise NotImplementedError("write your pallas kernel here")


# speedup vs baseline: 9.3375x; 9.3375x over previous
import functools

import jax
import jax.numpy as jnp
from jax import lax
from jax.experimental import pallas as pl
from jax.experimental.pallas import tpu as pltpu
from jax.experimental.pallas import tpu_sc as plsc

_B, _H, _W, _C = 4, 112, 112, 192
_NB = _H * _W * _C
_N = _B * _NB
_OB = (2 * _H) * (2 * _W) * _C
_OUT_WORDS = _B * _OB

_CH = 14 * 131072
_K = 6
_LAST = _OB - (_K - 1) * _CH
_TRASH = 512
_NSUB = 16
_PT = _NB // _NSUB
_WIN = 3072
_NWIN = _PT // _WIN
_ZW = 2048
_STG = 2048
_NROUND = (_B * _K) // 2


def _dest_body(mask_ref, dest_ref):
    m = mask_ref[...]
    ch = lax.broadcasted_iota(jnp.int32, m.shape, 1)
    dest_ref[...] = m - m % _C + ch


def _compute_dest(mask):
    m2 = mask.reshape(_B * _H * _W, _C)
    dest = pl.pallas_call(
        _dest_body,
        out_shape=jax.ShapeDtypeStruct(m2.shape, jnp.int32),
        grid=(49,),
        in_specs=[pl.BlockSpec((1024, _C), lambda i: (i, 0))],
        out_specs=pl.BlockSpec((1024, _C), lambda i: (i, 0)),
    )(m2)
    return dest.reshape(_N)


_mesh = plsc.VectorSubcoreMesh(core_axis_name="c", subcore_axis_name="s")


@functools.partial(
    pl.kernel,
    out_type=jax.ShapeDtypeStruct((_OUT_WORDS,), jnp.float32),
    mesh=_mesh,
    scratch_types=[
        pltpu.VMEM_SHARED((_CH + _TRASH,), jnp.float32),
        pltpu.VMEM((_ZW,), jnp.float32),
        pltpu.VMEM((_WIN,), jnp.int32),
        pltpu.VMEM((_WIN,), jnp.float32),
        pltpu.VMEM((_WIN,), jnp.int32),
        pltpu.VMEM((_STG,), jnp.float32),
    ],
)
def _sc_scatter(dest_hbm, upd_hbm, out_hbm, acc_sh, zero_v, dwin_v, uwin_v,
                idx_v, stage_v):
    core = lax.axis_index("c")
    sub = lax.axis_index("s")

    @pl.loop(0, _ZW // 16)
    def _(i):
        zero_v[pl.ds(i * 16, 16)] = jnp.zeros((16,), jnp.float32)

    @pl.loop(0, _NROUND)
    def _(r):
        g = r * 2 + core
        b = g // _K
        k = g % _K

        zbase = sub * (_CH // _NSUB)

        @pl.loop(0, _CH // _NSUB // _ZW)
        def _(j):
            zoff = pl.multiple_of(zbase + j * _ZW, 8)
            pltpu.sync_copy(zero_v, acc_sh.at[pl.ds(zoff, _ZW)])

        plsc.subcore_barrier()

        ibase = b * _NB + sub * _PT
        cbase = k * _CH

        @pl.loop(0, _NWIN)
        def _(w):
            off = pl.multiple_of(ibase + w * _WIN, 8)
            pltpu.sync_copy(dest_hbm.at[pl.ds(off, _WIN)], dwin_v)
            pltpu.sync_copy(upd_hbm.at[pl.ds(off, _WIN)], uwin_v)

            @pl.loop(0, _WIN // 16)
            def _(i):
                d = dwin_v[pl.ds(i * 16, 16)]
                rel = d - cbase
                inb = (rel >= 0) & (rel < _CH)
                tr = _CH + lax.bitwise_and(d, _TRASH - 1)
                idx_v[pl.ds(i * 16, 16)] = jnp.where(inb, rel, tr)

            pltpu.sync_copy(uwin_v, acc_sh.at[idx_v], add=True)

        plsc.subcore_barrier()

        valid = jnp.where(k == _K - 1, _LAST, _CH)
        share = valid // _NSUB
        obase = b * _OB + k * _CH + sub * share

        @pl.loop(0, share // _STG)
        def _(j):
            soff = pl.multiple_of(sub * share + j * _STG, 8)
            pltpu.sync_copy(acc_sh.at[pl.ds(soff, _STG)], stage_v)
            dsto = pl.multiple_of(obase + j * _STG, 8)
            pltpu.sync_copy(stage_v, out_hbm.at[pl.ds(dsto, _STG)])

        plsc.subcore_barrier()


def kernel(updates, mask):
    mask = mask.astype(jnp.int32)
    dest = _compute_dest(mask)
    upd = updates.reshape(_N)
    out = _sc_scatter(dest, upd)
    return out.reshape(_B, 2 * _H, 2 * _W, _C)


# speedup vs baseline: 13.7735x; 1.4751x over previous
import functools

import jax
import jax.numpy as jnp
from jax import lax
from jax.experimental import pallas as pl
from jax.experimental.pallas import tpu as pltpu
from jax.experimental.pallas import tpu_sc as plsc

_B, _H, _W, _C = 4, 112, 112, 192
_NB = _H * _W * _C
_N = _B * _NB
_OB = (2 * _H) * (2 * _W) * _C
_OUT_WORDS = _B * _OB

_CH = 15 * 131072
_K = 5
_LAST = _OB - (_K - 1) * _CH
_TRASH = 512
_NSUB = 16
_PT = _NB // _NSUB
_WIN = 1536
_NWIN = _PT // _WIN
_ZSH = _CH // _NSUB
_SSH = _CH // _NSUB
_LSH = _LAST // _NSUB
_NROUND = (_B * _K) // 2


def _dest_body(mask_ref, dest_ref):
    m = mask_ref[...]
    ch = lax.broadcasted_iota(jnp.int32, m.shape, 1)
    dest_ref[...] = m - m % _C + ch


def _compute_dest(mask):
    m2 = mask.reshape(_B * _H * _W, _C)
    dest = pl.pallas_call(
        _dest_body,
        out_shape=jax.ShapeDtypeStruct(m2.shape, jnp.int32),
        grid=(49,),
        in_specs=[pl.BlockSpec((1024, _C), lambda i: (i, 0))],
        out_specs=pl.BlockSpec((1024, _C), lambda i: (i, 0)),
    )(m2)
    return dest.reshape(_N)


_mesh = plsc.VectorSubcoreMesh(core_axis_name="c", subcore_axis_name="s")


@functools.partial(
    pl.kernel,
    out_type=jax.ShapeDtypeStruct((_OUT_WORDS,), jnp.float32),
    mesh=_mesh,
    scratch_types=[
        pltpu.VMEM_SHARED((_CH + _TRASH,), jnp.float32),
        pltpu.VMEM((_WIN,), jnp.int32),
        pltpu.VMEM((_WIN,), jnp.int32),
        pltpu.VMEM((_WIN,), jnp.float32),
        pltpu.VMEM((_WIN,), jnp.float32),
        pltpu.VMEM((_WIN,), jnp.int32),
        pltpu.SemaphoreType.DMA,
        pltpu.SemaphoreType.DMA,
        pltpu.SemaphoreType.DMA,
        pltpu.SemaphoreType.DMA,
    ],
)
def _sc_scatter(dest_hbm, upd_hbm, zero_hbm, out_hbm, acc_sh, d0, d1, u0, u1,
                idx_v, sd0, sd1, su0, su1):
    core = lax.axis_index("c")
    sub = lax.axis_index("s")

    @pl.loop(0, _NROUND)
    def _(r):
        g = r * 2 + core
        b = g // _K
        k = g % _K
        ibase = b * _NB + sub * _PT
        cbase = k * _CH

        zoff = pl.multiple_of(sub * _ZSH, 8)
        pltpu.sync_copy(zero_hbm, acc_sh.at[pl.ds(zoff, _ZSH)])
        plsc.subcore_barrier()

        def start_load(w, db, ub, sdb, sub_):
            off = pl.multiple_of(ibase + w * _WIN, 8)
            pltpu.async_copy(dest_hbm.at[pl.ds(off, _WIN)], db, sdb)
            pltpu.async_copy(upd_hbm.at[pl.ds(off, _WIN)], ub, sub_)

        def wait_load(db, ub, sdb, sub_):
            pltpu.make_async_copy(dest_hbm.at[pl.ds(0, _WIN)], db, sdb).wait()
            pltpu.make_async_copy(upd_hbm.at[pl.ds(0, _WIN)], ub, sub_).wait()

        def process(db, ub):
            @pl.loop(0, _WIN // 16)
            def _(i):
                d = db[pl.ds(i * 16, 16)]
                rel = d - cbase
                inb = plsc.bitcast(rel, jnp.uint32) < jnp.uint32(_CH)
                tr = _CH + (d & (_TRASH - 1))
                idx_v[pl.ds(i * 16, 16)] = jnp.where(inb, rel, tr)

            pltpu.sync_copy(ub, acc_sh.at[idx_v], add=True)

        start_load(0, d0, u0, sd0, su0)
        start_load(1, d1, u1, sd1, su1)

        @pl.loop(0, _NWIN // 2 - 1)
        def _(j):
            w = j * 2
            wait_load(d0, u0, sd0, su0)
            process(d0, u0)
            start_load(w + 2, d0, u0, sd0, su0)
            wait_load(d1, u1, sd1, su1)
            process(d1, u1)
            start_load(w + 3, d1, u1, sd1, su1)

        wait_load(d0, u0, sd0, su0)
        process(d0, u0)
        wait_load(d1, u1, sd1, su1)
        process(d1, u1)
        plsc.subcore_barrier()

        @pl.when(k < _K - 1)
        def _():
            soff = pl.multiple_of(sub * _SSH, 8)
            obase = pl.multiple_of(b * _OB + k * _CH + sub * _SSH, 8)
            pltpu.sync_copy(acc_sh.at[pl.ds(soff, _SSH)],
                            out_hbm.at[pl.ds(obase, _SSH)])

        @pl.when(k == _K - 1)
        def _():
            soff = pl.multiple_of(sub * _LSH, 8)
            obase = pl.multiple_of(b * _OB + k * _CH + sub * _LSH, 8)
            pltpu.sync_copy(acc_sh.at[pl.ds(soff, _LSH)],
                            out_hbm.at[pl.ds(obase, _LSH)])

        plsc.subcore_barrier()


def kernel(updates, mask):
    mask = mask.astype(jnp.int32)
    dest = _compute_dest(mask)
    upd = updates.reshape(_N)
    zeros = jnp.zeros((_ZSH,), jnp.float32)
    out = _sc_scatter(dest, upd, zeros)
    return out.reshape(_B, 2 * _H, 2 * _W, _C)


# speedup vs baseline: 13.7789x; 1.0004x over previous
import functools

import jax
import jax.numpy as jnp
from jax import lax
from jax.experimental import pallas as pl
from jax.experimental.pallas import tpu as pltpu
from jax.experimental.pallas import tpu_sc as plsc

_B, _H, _W, _C = 4, 112, 112, 192
_NB = _H * _W * _C
_N = _B * _NB
_OB = (2 * _H) * (2 * _W) * _C
_OUT_WORDS = _B * _OB

_TRASH = 16384
_CH = 15 * 131072 - _TRASH
_K = 5
_LAST = _OB - (_K - 1) * _CH
_NSUB = 16
_PT = _NB // _NSUB
_WIN = 1536
_NWIN = _PT // _WIN
_ZSH = _CH // _NSUB
_SSH = _CH // _NSUB
_LSH = _LAST // _NSUB
_NROUND = (_B * _K) // 2


def _dest_body(mask_ref, dest_ref):
    m = mask_ref[...]
    ch = lax.broadcasted_iota(jnp.int32, m.shape, 1)
    dest_ref[...] = m - m % _C + ch


def _compute_dest(mask):
    m2 = mask.reshape(_B * _H * _W, _C)
    dest = pl.pallas_call(
        _dest_body,
        out_shape=jax.ShapeDtypeStruct(m2.shape, jnp.int32),
        grid=(49,),
        in_specs=[pl.BlockSpec((1024, _C), lambda i: (i, 0))],
        out_specs=pl.BlockSpec((1024, _C), lambda i: (i, 0)),
    )(m2)
    return dest.reshape(_N)


_mesh = plsc.VectorSubcoreMesh(core_axis_name="c", subcore_axis_name="s")


@functools.partial(
    pl.kernel,
    out_type=jax.ShapeDtypeStruct((_OUT_WORDS,), jnp.float32),
    mesh=_mesh,
    scratch_types=[
        pltpu.VMEM_SHARED((_CH + _TRASH,), jnp.float32),
        pltpu.VMEM((_WIN,), jnp.int32),
        pltpu.VMEM((_WIN,), jnp.int32),
        pltpu.VMEM((_WIN,), jnp.float32),
        pltpu.VMEM((_WIN,), jnp.float32),
        pltpu.VMEM((_WIN,), jnp.int32),
        pltpu.SemaphoreType.DMA,
        pltpu.SemaphoreType.DMA,
        pltpu.SemaphoreType.DMA,
        pltpu.SemaphoreType.DMA,
    ],
)
def _sc_scatter(dest_hbm, upd_hbm, zero_hbm, out_hbm, acc_sh, d0, d1, u0, u1,
                idx_v, sd0, sd1, su0, su1):
    core = lax.axis_index("c")
    sub = lax.axis_index("s")

    @pl.loop(0, _NROUND)
    def _(r):
        g = r * 2 + core
        b = g // _K
        k = g % _K
        ibase = b * _NB + sub * _PT
        cbase = k * _CH

        zoff = pl.multiple_of(sub * _ZSH, 8)
        pltpu.sync_copy(zero_hbm.at[pl.ds(zoff, _ZSH)],
                        acc_sh.at[pl.ds(zoff, _ZSH)])
        plsc.subcore_barrier()

        def start_load(w, db, ub, sdb, sub_):
            off = pl.multiple_of(ibase + w * _WIN, 8)
            pltpu.async_copy(dest_hbm.at[pl.ds(off, _WIN)], db, sdb)
            pltpu.async_copy(upd_hbm.at[pl.ds(off, _WIN)], ub, sub_)

        def wait_load(db, ub, sdb, sub_):
            pltpu.make_async_copy(dest_hbm.at[pl.ds(0, _WIN)], db, sdb).wait()
            pltpu.make_async_copy(upd_hbm.at[pl.ds(0, _WIN)], ub, sub_).wait()

        def process(db, ub):
            @pl.loop(0, _WIN // 16)
            def _(i):
                d = db[pl.ds(i * 16, 16)]
                rel = d - cbase
                inb = plsc.bitcast(rel, jnp.uint32) < jnp.uint32(_CH)
                tr = _CH + (d & (_TRASH - 1))
                idx_v[pl.ds(i * 16, 16)] = jnp.where(inb, rel, tr)

            pltpu.sync_copy(ub, acc_sh.at[idx_v], add=True)

        start_load(0, d0, u0, sd0, su0)
        start_load(1, d1, u1, sd1, su1)

        @pl.loop(0, _NWIN // 2 - 1)
        def _(j):
            w = j * 2
            wait_load(d0, u0, sd0, su0)
            process(d0, u0)
            start_load(w + 2, d0, u0, sd0, su0)
            wait_load(d1, u1, sd1, su1)
            process(d1, u1)
            start_load(w + 3, d1, u1, sd1, su1)

        wait_load(d0, u0, sd0, su0)
        process(d0, u0)
        wait_load(d1, u1, sd1, su1)
        process(d1, u1)
        plsc.subcore_barrier()

        @pl.when(k < _K - 1)
        def _():
            soff = pl.multiple_of(sub * _SSH, 8)
            obase = pl.multiple_of(b * _OB + k * _CH + sub * _SSH, 8)
            pltpu.sync_copy(acc_sh.at[pl.ds(soff, _SSH)],
                            out_hbm.at[pl.ds(obase, _SSH)])

        @pl.when(k == _K - 1)
        def _():
            soff = pl.multiple_of(sub * _LSH, 8)
            obase = pl.multiple_of(b * _OB + k * _CH + sub * _LSH, 8)
            pltpu.sync_copy(acc_sh.at[pl.ds(soff, _LSH)],
                            out_hbm.at[pl.ds(obase, _LSH)])

        plsc.subcore_barrier()


def kernel(updates, mask):
    mask = mask.astype(jnp.int32)
    dest = _compute_dest(mask)
    upd = updates.reshape(_N)
    zeros = jnp.zeros((_CH,), jnp.float32)
    out = _sc_scatter(dest, upd, zeros)
    return out.reshape(_B, 2 * _H, 2 * _W, _C)
